# baseline (speedup 1.0000x reference)
import functools

import jax
import jax.numpy as jnp
from jax import lax
from jax.experimental import pallas as pl
from jax.experimental.pallas import tpu as pltpu
from jax.experimental.pallas import tpu_sc as plsc

B, T, N, H = 4, 16, 10000, 64


def _copy_body(sp_ref, stack_ref, hid_ref, out_ref):
    b = pl.program_id(0)
    t = pl.program_id(1)
    tgt = sp_ref[b] + 1

    @pl.when(t == tgt)
    def _():
        out_ref[...] = hid_ref[...]

    @pl.when(t != tgt)
    def _():
        out_ref[...] = stack_ref[...]


def _ptr_body(sp_hbm, hint_hbm, out_hbm, sp_v, hint_v, res_v):
    cid = lax.axis_index("c")
    sid = lax.axis_index("s")

    @pl.when((cid == 0) & (sid == 0))
    def _():
        pltpu.sync_copy(sp_hbm.at[0], sp_v.at[pl.ds(0, B)])
        pltpu.sync_copy(hint_hbm.at[0], hint_v.at[pl.ds(0, 3 * B)])
        iota = lax.iota(jnp.int32, 16)
        idx = lax.bitwise_and(iota, jnp.full((16,), B - 1, jnp.int32))
        zero = jnp.zeros((16,), jnp.int32)
        one = jnp.full((16,), 1, jnp.int32)
        two = jnp.full((16,), 2, jnp.int32)
        base = idx * 3
        sp16 = sp_v[...]
        hv = hint_v[...]
        take = lambda x, i: x.at[i].get(mode='promise_in_bounds')
        spv = take(sp16, idx)
        a0 = take(hv, base)
        a1 = take(hv, base + one)
        a2 = take(hv, base + two)
        ops = jnp.where(a0 >= a1,
                        jnp.where(a0 >= a2, zero, two),
                        jnp.where(a1 >= a2, one, two))
        res_v[...] = jnp.maximum(spv + ops - one, zero)
        pltpu.sync_copy(res_v.at[pl.ds(0, B)], out_hbm.at[0])


@functools.partial(
    pl.kernel,
    out_type=jax.ShapeDtypeStruct((1, B), jnp.int32),
    mesh=plsc.VectorSubcoreMesh(core_axis_name="c", subcore_axis_name="s"),
    scratch_types=[
        pltpu.VMEM((16,), jnp.int32),
        pltpu.VMEM((16,), jnp.float32),
        pltpu.VMEM((16,), jnp.int32),
    ],
)
def _ptr_kernel(sp_hbm, hint_hbm, out_hbm, sp_v, hint_v, res_v):
    _ptr_body(sp_hbm, hint_hbm, out_hbm, sp_v, hint_v, res_v)


def kernel(stack, stack_pointers, hint_preds, hiddens, graph_fts):
    del graph_fts
    sp_flat = jnp.reshape(stack_pointers, (B,))
    stack_t = jnp.transpose(stack, (0, 1, 3, 2))
    hid_t = jnp.transpose(hiddens, (0, 1, 3, 2))

    def stack_idx(b, t, sp):
        tt = jnp.where(t == sp[b] + 1, t - 1, t)
        return (b, tt, 0, 0)

    grid_spec = pltpu.PrefetchScalarGridSpec(
        num_scalar_prefetch=1,
        grid=(B, T),
        in_specs=[
            pl.BlockSpec((1, 1, H, N), stack_idx),
            pl.BlockSpec((1, 1, H, N), lambda b, t, sp: (0, b, 0, 0)),
        ],
        out_specs=pl.BlockSpec((1, 1, H, N), lambda b, t, sp: (b, t, 0, 0)),
    )

    new_stack_t = pl.pallas_call(
        _copy_body,
        grid_spec=grid_spec,
        out_shape=jax.ShapeDtypeStruct((B, T, H, N), jnp.float32),
    )(sp_flat, stack_t, hid_t)

    new_ptrs = _ptr_kernel(stack_pointers, jnp.reshape(hint_preds, (1, 3 * B)))
    return (jnp.transpose(new_stack_t, (0, 1, 3, 2)), new_ptrs)
